# Initial kernel scaffold; baseline (speedup 1.0000x reference)
import jax
import jax.numpy as jnp
from jax.experimental import pallas as pl


def kernel(x, edge_index, edge_type, rel_embed, qualifier_ent, qualifier_rel, w_loop, w_in, w_out, w_rel, w_q, loop_rel, bias, gamma, beta):
    raise NotImplementedError("write your pallas kernel here")


# speedup vs baseline: 1.0292x; 1.0292x over previous
import functools

import jax
import jax.numpy as jnp
from jax.experimental import pallas as pl
from jax.experimental.pallas import tpu as pltpu


def _final_kernel(in_res_ref, out_res_ref, x_ref, lr_ref, w_loop_ref,
                  w_in_ref, w_out_ref, bias_ref, gamma_ref, beta_ref,
                  out_ref):
    x = x_ref[...]
    loop_res = (x * lr_ref[...]) @ w_loop_ref[...]
    s = (in_res_ref[...] @ w_in_ref[...] + out_res_ref[...] @ w_out_ref[...]
         + loop_res) * (1.0 / 3.0) + bias_ref[...]
    mean = jnp.mean(s, axis=0, keepdims=True)
    var = jnp.mean((s - mean) ** 2, axis=0, keepdims=True)
    y = gamma_ref[...] * (s - mean) * jax.lax.rsqrt(var + 1e-5) + beta_ref[...]
    out_ref[...] = jnp.tanh(y)


def kernel(x, edge_index, edge_type, rel_embed, qualifier_ent, qualifier_rel,
           w_loop, w_in, w_out, w_rel, w_q, loop_rel, bias, gamma, beta):
    num_ent = x.shape[0]
    D = x.shape[1]
    rel_full = jnp.concatenate([rel_embed, loop_rel], axis=0)
    ne = edge_index.shape[1] // 2

    def half(idx, etype, qe, qr, w):
        row, col = idx[0], idx[1]
        deg = jax.ops.segment_sum(jnp.ones_like(row, jnp.float32), row,
                                  num_segments=num_ent)
        dinv = jnp.where(deg > 0, deg ** -0.5, 0.0)
        norm = dinv[row] * dinv[col]
        qsum = (x[qe[0]] * rel_full[qr[0]] + x[qe[1]] * rel_full[qr[1]])
        qagg = qsum @ w_q
        rel_emb = 0.5 * rel_full[etype] + 0.5 * qagg
        m = x[col] * rel_emb * norm[:, None]
        return jax.ops.segment_sum(m, row, num_segments=num_ent)

    in_res = half(edge_index[:, :ne], edge_type[:ne], qualifier_ent[:, :ne],
                  qualifier_rel[:, :ne], w_in)
    out_res = half(edge_index[:, ne:], edge_type[ne:], qualifier_ent[:, ne:],
                   qualifier_rel[:, ne:], w_out)

    out = pl.pallas_call(
        _final_kernel,
        out_shape=jax.ShapeDtypeStruct((num_ent, D), jnp.float32),
    )(in_res, out_res, x, loop_rel, w_loop, w_in, w_out,
      bias.reshape(1, D), gamma.reshape(1, D), beta.reshape(1, D))

    return out, (rel_full @ w_rel)[:-1]


# speedup vs baseline: 3.5178x; 3.4178x over previous
import functools

import jax
import jax.numpy as jnp
from jax import lax
from jax.experimental import pallas as pl
from jax.experimental.pallas import tpu as pltpu
from jax.experimental.pallas import tpu_sc as plsc

N = 10000
E = 320000
D = 128
RF = 401
RPAD = 416
NS = 16
NW = 32
EPW = E // NW
C = 80
CPT = EPW // C
NPAD = 10240
RPT = NPAD // NS


def _tile_ids():
    c = lax.axis_index("c")
    s = lax.axis_index("s")
    return c, s, (c * NS + s) * EPW


def _sca_body(x_hbm, rel_hbm, q6_hbm, out_hbm,
              q6_v, i0_v, i2_v, xa_v, ra_v, sem):
    c, s, base = _tile_ids()
    wid = c * NS + s

    @pl.loop(0, CPT)
    def _pa(k):
        off = base + k * C
        pltpu.sync_copy(q6_hbm.at[wid * CPT + k], q6_v)

        @pl.loop(0, C // 16)
        def _ia(g):
            sl = pl.ds(g * 16, 16)
            i0_v[sl] = q6_v[0, sl]
            i2_v[sl] = q6_v[2, sl]

        ga = pltpu.async_copy(x_hbm.at[i0_v], xa_v, sem)
        gc = pltpu.async_copy(rel_hbm.at[i2_v], ra_v, sem)
        ga.wait()
        gc.wait()

        @pl.loop(0, C)
        def _rows(e):
            for j in range(8):
                sl = pl.ds(16 * j, 16)
                ra_v[e, sl] = xa_v[e, sl] * ra_v[e, sl]

        pltpu.sync_copy(ra_v, out_hbm.at[pl.ds(off, C)])


def _scb_body(x_hbm, rel_hbm, q6_hbm, prev_hbm, out_hbm,
              q6_v, i1_v, i3_v, xa_v, ra_v, rb_v, sem):
    c, s, base = _tile_ids()
    wid = c * NS + s

    @pl.loop(0, CPT)
    def _pb(k):
        off = base + k * C
        pltpu.sync_copy(q6_hbm.at[wid * CPT + k], q6_v)

        @pl.loop(0, C // 16)
        def _ib(g):
            sl = pl.ds(g * 16, 16)
            i1_v[sl] = q6_v[1, sl]
            i3_v[sl] = q6_v[3, sl]

        ga = pltpu.async_copy(x_hbm.at[i1_v], xa_v, sem)
        gc = pltpu.async_copy(rel_hbm.at[i3_v], ra_v, sem)
        pltpu.sync_copy(prev_hbm.at[pl.ds(off, C)], rb_v)
        ga.wait()
        gc.wait()

        @pl.loop(0, C)
        def _rows(e):
            for j in range(8):
                sl = pl.ds(16 * j, 16)
                ra_v[e, sl] = xa_v[e, sl] * ra_v[e, sl] + rb_v[e, sl]

        pltpu.sync_copy(ra_v, out_hbm.at[pl.ds(off, C)])


def _scc_body(rel_hbm, q6_hbm, relp_hbm, q6_v, i5_v, xa_v, sem):
    c, s, base = _tile_ids()
    wid = c * NS + s

    @pl.loop(0, CPT)
    def _pc(k):
        off = base + k * C
        pltpu.sync_copy(q6_hbm.at[wid * CPT + k], q6_v)

        @pl.loop(0, C // 16)
        def _ic(g):
            sl = pl.ds(g * 16, 16)
            i5_v[sl] = q6_v[5, sl]

        pltpu.async_copy(rel_hbm.at[i5_v], xa_v, sem).wait()
        pltpu.sync_copy(xa_v, relp_hbm.at[pl.ds(off, C)])


def _scd_body(q6_hbm, deg_hbm, q6_v, i4_v, ones_v, deg_sp, sem):
    c, s, base = _tile_ids()
    wid = c * NS + s

    @pl.loop(0, C)
    def _z(r):
        ones_v[r, :] = jnp.zeros((16,), jnp.float32)

    for t in range(RPT // C):
        pltpu.sync_copy(ones_v, deg_sp.at[pl.ds(s * RPT + t * C, C)])

    @pl.loop(0, C)
    def _o(r):
        ones_v[r, :] = jnp.ones((16,), jnp.float32)

    plsc.subcore_barrier()

    @pl.loop(0, CPT)
    def _pd(k):
        pltpu.sync_copy(q6_hbm.at[wid * CPT + k], q6_v)

        @pl.loop(0, C // 16)
        def _id(g):
            sl = pl.ds(g * 16, 16)
            i4_v[sl] = q6_v[4, sl]

        pltpu.sync_copy(ones_v, deg_sp.at[i4_v], add=True)

    plsc.subcore_barrier()
    pltpu.sync_copy(deg_sp.at[pl.ds(s * RPT, RPT)],
                    deg_hbm.at[pl.ds(c * NPAD + s * RPT, RPT)])


def _sce_body(xs_hbm, qplus_hbm, q3_hbm, res_hbm,
              q3_v, ic_v, ir_v, xj_v, qa_v, res_sp, sem):
    c, s, base = _tile_ids()
    wid = c * NS + s

    @pl.loop(0, C)
    def _z(r):
        for j in range(8):
            xj_v[r, pl.ds(16 * j, 16)] = jnp.zeros((16,), jnp.float32)

    for t in range(RPT // C):
        pltpu.sync_copy(xj_v, res_sp.at[pl.ds(s * RPT + t * C, C)])

    plsc.subcore_barrier()

    @pl.loop(0, CPT)
    def _pe(k):
        off = base + k * C
        pltpu.sync_copy(q3_hbm.at[wid * CPT + k], q3_v)

        @pl.loop(0, C // 16)
        def _ie(g):
            sl = pl.ds(g * 16, 16)
            ic_v[sl] = q3_v[0, sl] + c * N
            ir_v[sl] = q3_v[2, sl]

        gx = pltpu.async_copy(xs_hbm.at[ic_v], xj_v, sem)
        pltpu.sync_copy(qplus_hbm.at[pl.ds(off, C)], qa_v)
        gx.wait()

        @pl.loop(0, C)
        def _rows(e):
            for j in range(8):
                sl = pl.ds(16 * j, 16)
                qa_v[e, sl] = qa_v[e, sl] * xj_v[e, sl]

        pltpu.sync_copy(qa_v, res_sp.at[ir_v], add=True)

    plsc.subcore_barrier()
    pltpu.sync_copy(res_sp.at[pl.ds(s * RPT, RPT)],
                    res_hbm.at[pl.ds(c * NPAD + s * RPT, RPT)])


def _dinv_body(deg_ref, x_ref, dinv_ref, xs_ref):
    d = deg_ref[...][:, 0]
    dinv = jnp.where(d > 0, lax.rsqrt(jnp.maximum(d, 1e-30)), 0.0)
    dinv_ref[...] = dinv
    x = x_ref[...]
    xs_ref[:N, :] = x * (0.5 * dinv[:N])[:, None]
    xs_ref[N:, :] = x * (0.5 * dinv[NPAD:NPAD + N])[:, None]


def _mm_body(a_ref, w_ref, r_ref, o_ref):
    o_ref[...] = jnp.dot(a_ref[...], w_ref[...],
                         preferred_element_type=jnp.float32) + r_ref[...]


def _final_body(in_ref, out_ref, din_ref, dout_ref, x_ref, lr_ref,
                w_loop_ref, w_in_ref, w_out_ref, rel_ref, w_rel_ref,
                bias_ref, gamma_ref, beta_ref, y_ref, relw_ref):
    x = x_ref[...]
    loop_res = (x * lr_ref[...]) @ w_loop_ref[...]
    a = (din_ref[...] * in_ref[...]) @ w_in_ref[...]
    b = (dout_ref[...] * out_ref[...]) @ w_out_ref[...]
    sm = (a + b + loop_res) * (1.0 / 3.0) + bias_ref[...]
    mean = jnp.mean(sm, axis=0, keepdims=True)
    var = jnp.mean((sm - mean) ** 2, axis=0, keepdims=True)
    y = gamma_ref[...] * (sm - mean) * lax.rsqrt(var + 1e-5) + beta_ref[...]
    y_ref[...] = jnp.tanh(y)
    relw_ref[...] = rel_ref[...] @ w_rel_ref[...]


def kernel(x, edge_index, edge_type, rel_embed, qualifier_ent, qualifier_rel,
           w_loop, w_in, w_out, w_rel, w_q, loop_rel, bias, gamma, beta):
    rel_full = jnp.concatenate([rel_embed, loop_rel], axis=0)
    rel_pad = jnp.concatenate(
        [rel_full, jnp.zeros((RPAD - RF, D), jnp.float32)], axis=0)
    q6 = jnp.stack([qualifier_ent[0], qualifier_ent[1],
                    qualifier_rel[0], qualifier_rel[1], edge_index[0],
                    edge_type])
    q6 = q6.reshape(6, E // C, C).transpose(1, 0, 2)
    q3 = jnp.stack([edge_index[1], edge_type, edge_index[0]])
    q3 = q3.reshape(3, E // C, C).transpose(1, 0, 2)

    mesh = plsc.VectorSubcoreMesh(core_axis_name="c", subcore_axis_name="s")
    ek = functools.partial(pl.kernel, mesh=mesh)
    edf32 = jax.ShapeDtypeStruct((E, D), jnp.float32)
    idx_t = pltpu.VMEM((C,), jnp.int32)
    row_t = pltpu.VMEM((C, D), jnp.float32)
    q6_t = pltpu.VMEM((6, C), jnp.int32)

    qsum_a = ek(out_type=edf32,
                scratch_types=[q6_t, idx_t, idx_t, row_t, row_t,
                               pltpu.SemaphoreType.DMA])(_sca_body)(
        x, rel_pad, q6)
    qsum = ek(out_type=edf32,
              scratch_types=[q6_t, idx_t, idx_t, row_t, row_t, row_t,
                             pltpu.SemaphoreType.DMA])(_scb_body)(
        x, rel_pad, q6, qsum_a)
    relp = ek(out_type=edf32,
              scratch_types=[q6_t, idx_t, row_t,
                             pltpu.SemaphoreType.DMA])(_scc_body)(
        rel_pad, q6)
    deg = ek(out_type=jax.ShapeDtypeStruct((2 * NPAD, 16), jnp.float32),
             scratch_types=[q6_t, idx_t,
                            pltpu.VMEM((C, 16), jnp.float32),
                            pltpu.VMEM_SHARED((NPAD, 16), jnp.float32),
                            pltpu.SemaphoreType.DMA])(_scd_body)(q6)

    half = (jnp.arange(E) >= E // 2).astype(jnp.int32)
    degx = jnp.zeros((2 * NPAD,), jnp.float32).at[
        edge_index[0] + half * NPAD].add(1.0)
    deg = jnp.broadcast_to(degx[:, None], (2 * NPAD, 16))

    dinv, xs = pl.pallas_call(
        _dinv_body,
        out_shape=(jax.ShapeDtypeStruct((2 * NPAD,), jnp.float32),
                   jax.ShapeDtypeStruct((2 * N, D), jnp.float32)),
    )(deg, x)

    qplus = pl.pallas_call(
        _mm_body,
        grid=(E // 3200,),
        in_specs=[pl.BlockSpec((3200, D), lambda i: (i, 0)),
                  pl.BlockSpec((D, D), lambda i: (0, 0)),
                  pl.BlockSpec((3200, D), lambda i: (i, 0))],
        out_specs=pl.BlockSpec((3200, D), lambda i: (i, 0)),
        out_shape=jax.ShapeDtypeStruct((E, D), jnp.float32),
    )(qsum, w_q, relp)

    v = xs[edge_index[1] + half * N] * qplus
    res = jnp.zeros((2 * NPAD, D), jnp.float32).at[
        edge_index[0] + half * NPAD].add(v)

    y, relw = pl.pallas_call(
        _final_body,
        out_shape=(jax.ShapeDtypeStruct((N, D), jnp.float32),
                   jax.ShapeDtypeStruct((RF, D), jnp.float32)),
    )(res[:N], res[NPAD:NPAD + N], dinv[:N, None], dinv[NPAD:NPAD + N, None],
      x, loop_rel, w_loop, w_in, w_out, rel_full, w_rel,
      bias.reshape(1, D), gamma.reshape(1, D), beta.reshape(1, D))

    return y, relw[:-1]


# speedup vs baseline: 4.3825x; 1.2458x over previous
import functools

import jax
import jax.numpy as jnp
from jax import lax
from jax.experimental import pallas as pl
from jax.experimental.pallas import tpu as pltpu
from jax.experimental.pallas import tpu_sc as plsc

N = 10000
E = 320000
D = 128
RF = 401
RPAD = 416
NS = 16
NW = 32
EPW = E // NW
C = 80
CPT = EPW // C
NPAD = 10240
RPT = NPAD // NS


def _tile_ids():
    c = lax.axis_index("c")
    s = lax.axis_index("s")
    return c, s, (c * NS + s) * EPW


def _sca_body(x_hbm, rel_hbm, q6_hbm, out_hbm,
              q6_v, i0_v, i2_v, xa_v, ra_v, sem):
    c, s, base = _tile_ids()
    wid = c * NS + s

    @pl.loop(0, CPT)
    def _pa(k):
        off = base + k * C
        pltpu.sync_copy(q6_hbm.at[wid * CPT + k], q6_v)

        @pl.loop(0, C // 16)
        def _ia(g):
            sl = pl.ds(g * 16, 16)
            i0_v[sl] = q6_v[0, sl]
            i2_v[sl] = q6_v[2, sl]

        ga = pltpu.async_copy(x_hbm.at[i0_v], xa_v, sem)
        gc = pltpu.async_copy(rel_hbm.at[i2_v], ra_v, sem)
        ga.wait()
        gc.wait()

        @pl.loop(0, C)
        def _rows(e):
            for j in range(8):
                sl = pl.ds(16 * j, 16)
                ra_v[e, sl] = xa_v[e, sl] * ra_v[e, sl]

        pltpu.sync_copy(ra_v, out_hbm.at[pl.ds(off, C)])


def _scb_body(x_hbm, rel_hbm, q6_hbm, prev_hbm, out_hbm,
              q6_v, i1_v, i3_v, xa_v, ra_v, rb_v, sem):
    c, s, base = _tile_ids()
    wid = c * NS + s

    @pl.loop(0, CPT)
    def _pb(k):
        off = base + k * C
        pltpu.sync_copy(q6_hbm.at[wid * CPT + k], q6_v)

        @pl.loop(0, C // 16)
        def _ib(g):
            sl = pl.ds(g * 16, 16)
            i1_v[sl] = q6_v[1, sl]
            i3_v[sl] = q6_v[3, sl]

        ga = pltpu.async_copy(x_hbm.at[i1_v], xa_v, sem)
        gc = pltpu.async_copy(rel_hbm.at[i3_v], ra_v, sem)
        pltpu.sync_copy(prev_hbm.at[pl.ds(off, C)], rb_v)
        ga.wait()
        gc.wait()

        @pl.loop(0, C)
        def _rows(e):
            for j in range(8):
                sl = pl.ds(16 * j, 16)
                ra_v[e, sl] = xa_v[e, sl] * ra_v[e, sl] + rb_v[e, sl]

        pltpu.sync_copy(ra_v, out_hbm.at[pl.ds(off, C)])


def _scc_body(rel_hbm, q6_hbm, relp_hbm, q6_v, i5_v, xa_v, sem):
    c, s, base = _tile_ids()
    wid = c * NS + s

    @pl.loop(0, CPT)
    def _pc(k):
        off = base + k * C
        pltpu.sync_copy(q6_hbm.at[wid * CPT + k], q6_v)

        @pl.loop(0, C // 16)
        def _ic(g):
            sl = pl.ds(g * 16, 16)
            i5_v[sl] = q6_v[5, sl]

        pltpu.async_copy(rel_hbm.at[i5_v], xa_v, sem).wait()
        pltpu.sync_copy(xa_v, relp_hbm.at[pl.ds(off, C)])


def _scd_body(q6_hbm, deg_hbm, q6_v, i4_v, ones_v, deg_sp, sem):
    c, s, base = _tile_ids()
    wid = c * NS + s

    @pl.loop(0, C)
    def _z(r):
        ones_v[r, :] = jnp.zeros((16,), jnp.float32)

    for t in range(RPT // C):
        pltpu.sync_copy(ones_v, deg_sp.at[pl.ds(s * RPT + t * C, C)])

    @pl.loop(0, C)
    def _o(r):
        ones_v[r, :] = jnp.ones((16,), jnp.float32)

    plsc.subcore_barrier()

    @pl.loop(0, CPT)
    def _pd(k):
        pltpu.sync_copy(q6_hbm.at[wid * CPT + k], q6_v)

        @pl.loop(0, C // 16)
        def _id(g):
            sl = pl.ds(g * 16, 16)
            i4_v[sl] = q6_v[4, sl]

        pltpu.sync_copy(ones_v, deg_sp.at[i4_v], add=True)

    plsc.subcore_barrier()
    pltpu.sync_copy(deg_sp.at[pl.ds(s * RPT, RPT)],
                    deg_hbm.at[pl.ds(c * NPAD + s * RPT, RPT)])


def _sce_body(xs_hbm, qplus_hbm, q3_hbm, res_hbm,
              q3_v, ic_v, ir_v, xj_v, qa_v, res_sp, sem):
    c, s, base = _tile_ids()
    wid = c * NS + s

    @pl.loop(0, C)
    def _z(r):
        for j in range(8):
            xj_v[r, pl.ds(16 * j, 16)] = jnp.zeros((16,), jnp.float32)

    for t in range(RPT // C):
        pltpu.sync_copy(xj_v, res_sp.at[pl.ds(s * RPT + t * C, C)])

    plsc.subcore_barrier()

    @pl.loop(0, CPT)
    def _pe(k):
        off = base + k * C
        pltpu.sync_copy(q3_hbm.at[wid * CPT + k], q3_v)

        @pl.loop(0, C // 16)
        def _ie(g):
            sl = pl.ds(g * 16, 16)
            ic_v[sl] = q3_v[0, sl] + c * N
            ir_v[sl] = q3_v[2, sl]

        gx = pltpu.async_copy(xs_hbm.at[ic_v], xj_v, sem)
        pltpu.sync_copy(qplus_hbm.at[pl.ds(off, C)], qa_v)
        gx.wait()

        @pl.loop(0, C)
        def _rows(e):
            for j in range(8):
                sl = pl.ds(16 * j, 16)
                qa_v[e, sl] = qa_v[e, sl] * xj_v[e, sl]

        pltpu.sync_copy(qa_v, res_sp.at[ir_v], add=True)

    plsc.subcore_barrier()
    pltpu.sync_copy(res_sp.at[pl.ds(s * RPT, RPT)],
                    res_hbm.at[pl.ds(c * NPAD + s * RPT, RPT)])


def _dinv_body(deg_ref, x_ref, dinv_ref, xs_ref):
    d = deg_ref[...][:, 0]
    dinv = jnp.where(d > 0, lax.rsqrt(jnp.maximum(d, 1e-30)), 0.0)
    dinv_ref[...] = dinv
    x = x_ref[...]
    xs_ref[:N, :] = x * (0.5 * dinv[:N])[:, None]
    xs_ref[N:, :] = x * (0.5 * dinv[NPAD:NPAD + N])[:, None]


def _mm_body(a_ref, w_ref, r_ref, o_ref):
    o_ref[...] = jnp.dot(a_ref[...], w_ref[...],
                         preferred_element_type=jnp.float32) + r_ref[...]


def _final_body(in_ref, out_ref, din_ref, dout_ref, x_ref, lr_ref,
                w_loop_ref, w_in_ref, w_out_ref, rel_ref, w_rel_ref,
                bias_ref, gamma_ref, beta_ref, y_ref, relw_ref):
    x = x_ref[...]
    loop_res = (x * lr_ref[...]) @ w_loop_ref[...]
    a = (din_ref[...] * in_ref[...]) @ w_in_ref[...]
    b = (dout_ref[...] * out_ref[...]) @ w_out_ref[...]
    sm = (a + b + loop_res) * (1.0 / 3.0) + bias_ref[...]
    mean = jnp.mean(sm, axis=0, keepdims=True)
    var = jnp.mean((sm - mean) ** 2, axis=0, keepdims=True)
    y = gamma_ref[...] * (sm - mean) * lax.rsqrt(var + 1e-5) + beta_ref[...]
    y_ref[...] = jnp.tanh(y)
    relw_ref[...] = rel_ref[...] @ w_rel_ref[...]


def kernel(x, edge_index, edge_type, rel_embed, qualifier_ent, qualifier_rel,
           w_loop, w_in, w_out, w_rel, w_q, loop_rel, bias, gamma, beta):
    rel_full = jnp.concatenate([rel_embed, loop_rel], axis=0)
    rel_pad = jnp.concatenate(
        [rel_full, jnp.zeros((RPAD - RF, D), jnp.float32)], axis=0)
    q6 = jnp.stack([qualifier_ent[0], qualifier_ent[1],
                    qualifier_rel[0], qualifier_rel[1], edge_index[0],
                    edge_type])
    q6 = q6.reshape(6, E // C, C).transpose(1, 0, 2)
    q3 = jnp.stack([edge_index[1], edge_type, edge_index[0]])
    q3 = q3.reshape(3, E // C, C).transpose(1, 0, 2)

    mesh = plsc.VectorSubcoreMesh(core_axis_name="c", subcore_axis_name="s")
    ek = functools.partial(pl.kernel, mesh=mesh)
    edf32 = jax.ShapeDtypeStruct((E, D), jnp.float32)
    idx_t = pltpu.VMEM((C,), jnp.int32)
    row_t = pltpu.VMEM((C, D), jnp.float32)
    q6_t = pltpu.VMEM((6, C), jnp.int32)

    qsum_a = ek(out_type=edf32,
                scratch_types=[q6_t, idx_t, idx_t, row_t, row_t,
                               pltpu.SemaphoreType.DMA])(_sca_body)(
        x, rel_pad, q6)
    qsum = ek(out_type=edf32,
              scratch_types=[q6_t, idx_t, idx_t, row_t, row_t, row_t,
                             pltpu.SemaphoreType.DMA])(_scb_body)(
        x, rel_pad, q6, qsum_a)
    relp = ek(out_type=edf32,
              scratch_types=[q6_t, idx_t, row_t,
                             pltpu.SemaphoreType.DMA])(_scc_body)(
        rel_pad, q6)
    deg = ek(out_type=jax.ShapeDtypeStruct((2 * NPAD, 16), jnp.float32),
             scratch_types=[q6_t, idx_t,
                            pltpu.VMEM((C, 16), jnp.float32),
                            pltpu.VMEM_SHARED((NPAD, 16), jnp.float32),
                            pltpu.SemaphoreType.DMA])(_scd_body)(q6)

    half = (jnp.arange(E) >= E // 2).astype(jnp.int32)
    degx = jnp.zeros((2 * NPAD,), jnp.float32).at[
        edge_index[0] + half * NPAD].add(1.0)
    deg = jnp.broadcast_to(degx[:, None], (2 * NPAD, 16))

    dinv, xs = pl.pallas_call(
        _dinv_body,
        out_shape=(jax.ShapeDtypeStruct((2 * NPAD,), jnp.float32),
                   jax.ShapeDtypeStruct((2 * N, D), jnp.float32)),
    )(deg, x)

    qplus = pl.pallas_call(
        _mm_body,
        grid=(E // 3200,),
        in_specs=[pl.BlockSpec((3200, D), lambda i: (i, 0)),
                  pl.BlockSpec((D, D), lambda i: (0, 0)),
                  pl.BlockSpec((3200, D), lambda i: (i, 0))],
        out_specs=pl.BlockSpec((3200, D), lambda i: (i, 0)),
        out_shape=jax.ShapeDtypeStruct((E, D), jnp.float32),
    )(qsum, w_q, relp)

    res = ek(out_type=jax.ShapeDtypeStruct((2 * NPAD, D), jnp.float32),
             scratch_types=[pltpu.VMEM((3, C), jnp.int32), idx_t, idx_t,
                            row_t, row_t,
                            pltpu.VMEM_SHARED((NPAD, D), jnp.float32),
                            pltpu.SemaphoreType.DMA])(_sce_body)(
        xs, qplus, q3)

    y, relw = pl.pallas_call(
        _final_body,
        out_shape=(jax.ShapeDtypeStruct((N, D), jnp.float32),
                   jax.ShapeDtypeStruct((RF, D), jnp.float32)),
    )(res[:N], res[NPAD:NPAD + N], dinv[:N, None], dinv[NPAD:NPAD + N, None],
      x, loop_rel, w_loop, w_in, w_out, rel_full, w_rel,
      bias.reshape(1, D), gamma.reshape(1, D), beta.reshape(1, D))

    return y, relw[:-1]
